# Initial kernel scaffold; baseline (speedup 1.0000x reference)
import jax
import jax.numpy as jnp
from jax.experimental import pallas as pl


def kernel(input_ids, llm_table, Wx, bx, Wy, by, Ww, bw, Wh, bh):
    raise NotImplementedError("write your pallas kernel here")


# speedup vs baseline: 2.1754x; 2.1754x over previous
import functools
import math

import jax
import jax.numpy as jnp
from jax import lax
from jax.experimental import pallas as pl
from jax.experimental.pallas import tpu as pltpu
from jax.experimental.pallas import tpu_sc as plsc

N_TOKEN = 32000
CANVAS = 128
SIN_DIM = 64
HALF = SIN_DIM // 2
D = 2048
ROWS = 4 * 2048

NC, NS, LANES = 2, 16, 16
NW = NC * NS
PER_W = ROWS // NW
CHUNK = 16
NCHUNK = PER_W // CHUNK

_SCALE = math.log(100.0) / (HALF - 1)


def _spec_table_body(wx, bx, wy, by, ww, bw, wh, bh, out_ref):
    col = lax.broadcasted_iota(jnp.int32, (CANVAS, SIN_DIM), 1)
    colh = jnp.where(col < HALF, col, col - HALF).astype(jnp.float32)
    freq = jnp.exp(colh * (-_SCALE))
    row0 = lax.broadcasted_iota(jnp.int32, (CANVAS, SIN_DIM), 0).astype(jnp.float32)
    for k, (w_ref, b_ref) in enumerate(((wx, bx), (wy, by), (ww, bw), (wh, bh))):
        arg = (row0 + float(k * CANVAS)) * freq
        s = jnp.where(col < HALF, jnp.sin(arg), jnp.cos(arg))
        blk = lax.dot_general(s, w_ref[...], (((1,), (1,)), ((), ())),
                              preferred_element_type=jnp.float32)
        out_ref[k * CANVAS:(k + 1) * CANVAS, :] = blk + b_ref[...]


def _build_spec_table(Wx, bx, Wy, by, Ww, bw, Wh, bh):
    return pl.pallas_call(
        _spec_table_body,
        out_shape=jax.ShapeDtypeStruct((4 * CANVAS, D), jnp.float32),
    )(Wx, bx.reshape(1, D), Wy, by.reshape(1, D),
      Ww, bw.reshape(1, D), Wh, bh.reshape(1, D))


def _sc_body(ids_hbm, llm_hbm, spec_hbm, out_hbm, idxv, buf, sem):
    wid = lax.axis_index("s") * NC + lax.axis_index("c")
    base = wid * PER_W

    def chunk(g, carry):
        gbase = base + g * CHUNK
        pltpu.sync_copy(ids_hbm.at[pl.ds(gbase, CHUNK)], idxv)
        v = idxv[...]
        diff = v - N_TOKEN
        idxv[...] = jnp.where(diff < 0, v, 0)
        pltpu.async_copy(llm_hbm.at[idxv], buf, sem).wait()
        for i in range(CHUNK):
            d_i = diff[i]

            @pl.when(d_i >= 0)
            def _():
                pltpu.sync_copy(spec_hbm.at[pl.ds(d_i, 1)],
                                buf.at[pl.ds(i, 1)])

        pltpu.sync_copy(buf, out_hbm.at[pl.ds(gbase, CHUNK)])
        return carry

    lax.fori_loop(0, NCHUNK, chunk, 0)


def _sc_gather(ids, llm_table, spec_table):
    mesh = plsc.VectorSubcoreMesh(core_axis_name="c", subcore_axis_name="s",
                                  num_cores=NC, num_subcores=NS)
    return pl.kernel(
        _sc_body,
        out_type=jax.ShapeDtypeStruct((ROWS, D), jnp.float32),
        mesh=mesh,
        scratch_types=[
            pltpu.VMEM((CHUNK,), jnp.int32),
            pltpu.VMEM((CHUNK, D), jnp.float32),
            pltpu.SemaphoreType.DMA,
        ],
    )(ids, llm_table, spec_table)


def kernel(input_ids, llm_table, Wx, bx, Wy, by, Ww, bw, Wh, bh):
    spec = _build_spec_table(Wx, bx, Wy, by, Ww, bw, Wh, bh)
    ids = input_ids.reshape(ROWS)
    out = _sc_gather(ids, llm_table, spec)
    return out.reshape(input_ids.shape[0], input_ids.shape[1], D)


# speedup vs baseline: 2.3437x; 1.0773x over previous
import functools
import math

import jax
import jax.numpy as jnp
from jax import lax
from jax.experimental import pallas as pl
from jax.experimental.pallas import tpu as pltpu
from jax.experimental.pallas import tpu_sc as plsc

N_TOKEN = 32000
CANVAS = 128
SIN_DIM = 64
HALF = SIN_DIM // 2
D = 2048
ROWS = 4 * 2048

NC, NS, LANES = 2, 16, 16
NW = NC * NS
PER_W = ROWS // NW
CHUNK = 32
NCHUNK = PER_W // CHUNK

_SCALE = math.log(100.0) / (HALF - 1)


def _spec_table_body(wx, bx, wy, by, ww, bw, wh, bh, out_ref):
    col = lax.broadcasted_iota(jnp.int32, (CANVAS, SIN_DIM), 1)
    colh = jnp.where(col < HALF, col, col - HALF).astype(jnp.float32)
    freq = jnp.exp(colh * (-_SCALE))
    row0 = lax.broadcasted_iota(jnp.int32, (CANVAS, SIN_DIM), 0).astype(jnp.float32)
    for k, (w_ref, b_ref) in enumerate(((wx, bx), (wy, by), (ww, bw), (wh, bh))):
        arg = (row0 + float(k * CANVAS)) * freq
        s = jnp.where(col < HALF, jnp.sin(arg), jnp.cos(arg))
        blk = lax.dot_general(s, w_ref[...], (((1,), (1,)), ((), ())),
                              preferred_element_type=jnp.float32)
        out_ref[k * CANVAS:(k + 1) * CANVAS, :] = blk + b_ref[...]


def _build_spec_table(Wx, bx, Wy, by, Ww, bw, Wh, bh):
    return pl.pallas_call(
        _spec_table_body,
        out_shape=jax.ShapeDtypeStruct((4 * CANVAS, D), jnp.float32),
    )(Wx, bx.reshape(1, D), Wy, by.reshape(1, D),
      Ww, bw.reshape(1, D), Wh, bh.reshape(1, D))


def _sc_body(ids_hbm, llm_hbm, spec_hbm, out_hbm, idxv, buf, sem):
    wid = lax.axis_index("s") * NC + lax.axis_index("c")
    base = wid * PER_W

    def chunk(g, carry):
        gbase = base + g * CHUNK
        pltpu.sync_copy(ids_hbm.at[pl.ds(gbase, CHUNK)], idxv)
        diffs = []
        for h in range(CHUNK // LANES):
            v = idxv[pl.ds(h * LANES, LANES)]
            dh = v - N_TOKEN
            diffs.append(dh)
            idxv[pl.ds(h * LANES, LANES)] = jnp.where(dh < 0, v, 0)
        pltpu.async_copy(llm_hbm.at[idxv], buf, sem).wait()
        for h, dh in enumerate(diffs):
            for i in range(LANES):
                d_i = dh[i]

                @pl.when(d_i >= 0)
                def _():
                    pltpu.sync_copy(spec_hbm.at[pl.ds(d_i, 1)],
                                    buf.at[pl.ds(h * LANES + i, 1)])

        pltpu.sync_copy(buf, out_hbm.at[pl.ds(gbase, CHUNK)])
        return carry

    lax.fori_loop(0, NCHUNK, chunk, 0)


def _sc_gather(ids, llm_table, spec_table):
    mesh = plsc.VectorSubcoreMesh(core_axis_name="c", subcore_axis_name="s",
                                  num_cores=NC, num_subcores=NS)
    return pl.kernel(
        _sc_body,
        out_type=jax.ShapeDtypeStruct((ROWS, D), jnp.float32),
        mesh=mesh,
        scratch_types=[
            pltpu.VMEM((CHUNK,), jnp.int32),
            pltpu.VMEM((CHUNK, D), jnp.float32),
            pltpu.SemaphoreType.DMA,
        ],
    )(ids, llm_table, spec_table)


def kernel(input_ids, llm_table, Wx, bx, Wy, by, Ww, bw, Wh, bh):
    spec = _build_spec_table(Wx, bx, Wy, by, Ww, bw, Wh, bh)
    ids = input_ids.reshape(ROWS)
    out = _sc_gather(ids, llm_table, spec)
    return out.reshape(input_ids.shape[0], input_ids.shape[1], D)


# speedup vs baseline: 2.7103x; 1.1564x over previous
import functools
import math

import jax
import jax.numpy as jnp
from jax import lax
from jax.experimental import pallas as pl
from jax.experimental.pallas import tpu as pltpu
from jax.experimental.pallas import tpu_sc as plsc

N_TOKEN = 32000
CANVAS = 128
SIN_DIM = 64
HALF = SIN_DIM // 2
D = 2048
ROWS = 4 * 2048

NC, NS, LANES = 2, 16, 16
NW = NC * NS
PER_W = ROWS // NW
CHUNK = 16
NCHUNK = PER_W // CHUNK
NPAIR = NCHUNK // 2

_SCALE = math.log(100.0) / (HALF - 1)


def _spec_table_body(wx, bx, wy, by, ww, bw, wh, bh, out_ref):
    col = lax.broadcasted_iota(jnp.int32, (CANVAS, SIN_DIM), 1)
    colh = jnp.where(col < HALF, col, col - HALF).astype(jnp.float32)
    freq = jnp.exp(colh * (-_SCALE))
    row0 = lax.broadcasted_iota(jnp.int32, (CANVAS, SIN_DIM), 0).astype(jnp.float32)
    for k, (w_ref, b_ref) in enumerate(((wx, bx), (wy, by), (ww, bw), (wh, bh))):
        arg = (row0 + float(k * CANVAS)) * freq
        s = jnp.where(col < HALF, jnp.sin(arg), jnp.cos(arg))
        blk = lax.dot_general(s, w_ref[...], (((1,), (1,)), ((), ())),
                              preferred_element_type=jnp.float32)
        out_ref[k * CANVAS:(k + 1) * CANVAS, :] = blk + b_ref[...]


def _build_spec_table(Wx, bx, Wy, by, Ww, bw, Wh, bh):
    return pl.pallas_call(
        _spec_table_body,
        out_shape=jax.ShapeDtypeStruct((4 * CANVAS, D), jnp.float32),
    )(Wx, bx.reshape(1, D), Wy, by.reshape(1, D),
      Ww, bw.reshape(1, D), Wh, bh.reshape(1, D))


def _sc_body(ids_hbm, llm_hbm, spec_hbm, out_hbm,
             idx0, idx1, df0, df1, buf0, buf1, gs0, gs1, ws0, ws1):
    wid = lax.axis_index("s") * NC + lax.axis_index("c")
    base = wid * PER_W
    slots = ((idx0, df0, buf0, gs0, ws0), (idx1, df1, buf1, gs1, ws1))

    def issue_gather(c, idx, df, buf, gs):
        gb = base + c * CHUNK
        pltpu.sync_copy(ids_hbm.at[pl.ds(gb, CHUNK)], idx)
        v = idx[...]
        d = v - N_TOKEN
        df[...] = d
        idx[...] = jnp.where(d < 0, v, 0)
        pltpu.async_copy(llm_hbm.at[idx], buf, gs)

    for s in range(2):
        idx, df, buf, gs, ws = slots[s]
        issue_gather(s, idx, df, buf, gs)

    def pair(g, carry):
        for s in range(2):
            idx, df, buf, gs, ws = slots[s]
            c = 2 * g + s
            pltpu.make_async_copy(llm_hbm.at[idx], buf, gs).wait()
            d = df[...]
            for i in range(CHUNK):
                d_i = d[i]

                @pl.when(d_i >= 0)
                def _():
                    pltpu.sync_copy(spec_hbm.at[pl.ds(d_i, 1)],
                                    buf.at[pl.ds(i, 1)])

            gb = base + c * CHUNK
            pltpu.async_copy(buf, out_hbm.at[pl.ds(gb, CHUNK)], ws)
            pltpu.make_async_copy(buf, out_hbm.at[pl.ds(gb, CHUNK)], ws).wait()

            @pl.when(g < NPAIR - 1)
            def _():
                issue_gather(c + 2, idx, df, buf, gs)

        return carry

    lax.fori_loop(0, NPAIR, pair, 0)


def _sc_gather(ids, llm_table, spec_table):
    mesh = plsc.VectorSubcoreMesh(core_axis_name="c", subcore_axis_name="s",
                                  num_cores=NC, num_subcores=NS)
    return pl.kernel(
        _sc_body,
        out_type=jax.ShapeDtypeStruct((ROWS, D), jnp.float32),
        mesh=mesh,
        scratch_types=[
            pltpu.VMEM((CHUNK,), jnp.int32),
            pltpu.VMEM((CHUNK,), jnp.int32),
            pltpu.VMEM((CHUNK,), jnp.int32),
            pltpu.VMEM((CHUNK,), jnp.int32),
            pltpu.VMEM((CHUNK, D), jnp.float32),
            pltpu.VMEM((CHUNK, D), jnp.float32),
            pltpu.SemaphoreType.DMA,
            pltpu.SemaphoreType.DMA,
            pltpu.SemaphoreType.DMA,
            pltpu.SemaphoreType.DMA,
        ],
    )(ids, llm_table, spec_table)


def kernel(input_ids, llm_table, Wx, bx, Wy, by, Ww, bw, Wh, bh):
    spec = _build_spec_table(Wx, bx, Wy, by, Ww, bw, Wh, bh)
    ids = input_ids.reshape(ROWS)
    out = _sc_gather(ids, llm_table, spec)
    return out.reshape(input_ids.shape[0], input_ids.shape[1], D)
